# baseline (speedup 1.0000x reference)
import jax
import jax.numpy as jnp
from jax.experimental import pallas as pl
from jax.experimental.pallas import tpu as pltpu

E = 8
N_GROUP = 4
TOPK_GROUP = 2
TOP_K = 2
GROUP_SIZE = E // N_GROUP
ROUTED_SCALING = 2.5
NEG = -1e30
N_SHARED_CHUNKS = 4


def _logits_kernel(x_ref, gw_ref, out_ref):
    out_ref[...] = jax.lax.dot_general(
        gw_ref[...], x_ref[...], (((1,), (1,)), ((), ())),
        preferred_element_type=jnp.float32)


def _routing_kernel(lg_ref, bias_ref, w_ref):
    sc = [1.0 / (1.0 + jnp.exp(-lg_ref[e])) for e in range(E)]
    sb = [sc[e] + bias_ref[e] for e in range(E)]
    g = [sb[GROUP_SIZE * gi] + sb[GROUP_SIZE * gi + 1] for gi in range(N_GROUP)]
    gmask = []
    for gi in range(N_GROUP):
        r = jnp.zeros_like(g[gi])
        for gj in range(N_GROUP):
            if gj == gi:
                continue
            beats = (g[gj] > g[gi]) if gj > gi else (g[gj] >= g[gi])
            r = r + beats.astype(jnp.float32)
        gmask.append(r < TOPK_GROUP)
    ms = [jnp.where(gmask[e // GROUP_SIZE], sb[e], NEG) for e in range(E)]
    wts = []
    for ei in range(E):
        r = jnp.zeros_like(ms[ei])
        for ej in range(E):
            if ej == ei:
                continue
            beats = (ms[ej] > ms[ei]) if ej > ei else (ms[ej] >= ms[ei])
            r = r + beats.astype(jnp.float32)
        sel = r < TOP_K
        wts.append(jnp.where(sel, sc[ei], 0.0))
    denom = wts[0]
    for e in range(1, E):
        denom = denom + wts[e]
    denom = denom + 1e-20
    ones = jnp.ones_like(wts[0])
    for c in range(N_SHARED_CHUNKS):
        w_ref[c] = ones
    for e in range(E):
        w_ref[N_SHARED_CHUNKS + e] = ROUTED_SCALING * (wts[e] / denom)


def _moe_kernel(x_ref, wt_ref, w_up_ref, w_down_ref, out_ref):
    step = pl.program_id(0)
    xb = x_ref[...]
    h = jax.lax.dot_general(
        xb, w_up_ref[0], (((1,), (1,)), ((), ())),
        preferred_element_type=jnp.float32)
    h = jnp.square(jnp.maximum(h, 0.0)).astype(jnp.bfloat16)
    y = jax.lax.dot_general(
        h, w_down_ref[0], (((1,), (1,)), ((), ())),
        preferred_element_type=jnp.float32)
    z = wt_ref[0] * y

    @pl.when(step == 0)
    def _():
        out_ref[...] = z

    @pl.when(step != 0)
    def _():
        out_ref[...] = out_ref[...] + z


def kernel(hidden_states, gate_w, e_score_correction_bias, w_up, w_down,
           shared_w_up, shared_w_down):
    t, d = hidden_states.shape
    e, d_ff, _ = w_up.shape
    d_ff_sh = shared_w_up.shape[0]
    assert e == E and d_ff_sh == N_SHARED_CHUNKS * d_ff
    n_steps = N_SHARED_CHUNKS + E

    x_bf = hidden_states.astype(jnp.bfloat16)
    gw_bf = gate_w.astype(jnp.bfloat16)

    logits_t = pl.pallas_call(
        _logits_kernel,
        out_shape=jax.ShapeDtypeStruct((E, t), jnp.float32),
    )(x_bf, gw_bf)

    tb = t // 128
    lg3 = logits_t.reshape(E, tb, 128)
    bias3 = e_score_correction_bias.reshape(E, 1, 1)
    w3 = pl.pallas_call(
        _routing_kernel,
        out_shape=jax.ShapeDtypeStruct((n_steps, tb, 128), jnp.float32),
    )(lg3, bias3)
    wt3 = w3.reshape(n_steps, t, 1)

    up_all = jnp.concatenate(
        [shared_w_up.reshape(N_SHARED_CHUNKS, d_ff, d), w_up],
        axis=0).astype(jnp.bfloat16)
    down_all = jnp.concatenate(
        [shared_w_down.reshape(d, N_SHARED_CHUNKS, d_ff).transpose(1, 0, 2),
         w_down],
        axis=0).astype(jnp.bfloat16)

    out = pl.pallas_call(
        _moe_kernel,
        grid=(n_steps,),
        in_specs=[
            pl.BlockSpec((t, d), lambda i: (0, 0)),
            pl.BlockSpec((1, t, 1), lambda i: (i, 0, 0)),
            pl.BlockSpec((1, d_ff, d), lambda i: (i, 0, 0)),
            pl.BlockSpec((1, d, d_ff), lambda i: (i, 0, 0)),
        ],
        out_specs=pl.BlockSpec((t, d), lambda i: (0, 0)),
        out_shape=jax.ShapeDtypeStruct((t, d), jnp.float32),
        compiler_params=pltpu.CompilerParams(
            dimension_semantics=("arbitrary",),
            vmem_limit_bytes=100 * 1024 * 1024,
        ),
    )(x_bf, wt3, up_all, down_all)
    return out


# speedup vs baseline: 1.4530x; 1.4530x over previous
import jax
import jax.numpy as jnp
from jax.experimental import pallas as pl
from jax.experimental.pallas import tpu as pltpu

E = 8
N_GROUP = 4
TOPK_GROUP = 2
TOP_K = 2
GROUP_SIZE = E // N_GROUP
ROUTED_SCALING = 2.5
NEG = -1e30
N_SHARED_CHUNKS = 4


def _logits_kernel(x_ref, gw_ref, out_ref):
    out_ref[...] = jax.lax.dot_general(
        gw_ref[...], x_ref[...], (((1,), (1,)), ((), ())),
        preferred_element_type=jnp.float32)


def _routing_kernel(lg_ref, bias_ref, w_ref):
    sc = [1.0 / (1.0 + jnp.exp(-lg_ref[e])) for e in range(E)]
    sb = [sc[e] + bias_ref[e] for e in range(E)]
    g = [sb[GROUP_SIZE * gi] + sb[GROUP_SIZE * gi + 1] for gi in range(N_GROUP)]
    gmask = []
    for gi in range(N_GROUP):
        r = jnp.zeros_like(g[gi])
        for gj in range(N_GROUP):
            if gj == gi:
                continue
            beats = (g[gj] > g[gi]) if gj > gi else (g[gj] >= g[gi])
            r = r + beats.astype(jnp.float32)
        gmask.append(r < TOPK_GROUP)
    ms = [jnp.where(gmask[e // GROUP_SIZE], sb[e], NEG) for e in range(E)]
    wts = []
    for ei in range(E):
        r = jnp.zeros_like(ms[ei])
        for ej in range(E):
            if ej == ei:
                continue
            beats = (ms[ej] > ms[ei]) if ej > ei else (ms[ej] >= ms[ei])
            r = r + beats.astype(jnp.float32)
        sel = r < TOP_K
        wts.append(jnp.where(sel, sc[ei], 0.0))
    denom = wts[0]
    for e in range(1, E):
        denom = denom + wts[e]
    denom = denom + 1e-20
    ones = jnp.ones_like(wts[0])
    for c in range(N_SHARED_CHUNKS):
        w_ref[c] = ones
    for e in range(E):
        w_ref[N_SHARED_CHUNKS + e] = ROUTED_SCALING * (wts[e] / denom)


def _moe_kernel(x_ref, wt_ref, up_ref, down_ref, sup_ref, sdown_ref, out_ref):
    step = pl.program_id(0)
    xb = x_ref[...]

    def mlp(wu, wd):
        h = jax.lax.dot_general(
            xb, wu.astype(jnp.bfloat16), (((1,), (1,)), ((), ())),
            preferred_element_type=jnp.float32)
        h = jnp.square(jnp.maximum(h, 0.0)).astype(jnp.bfloat16)
        return jax.lax.dot_general(
            h, wd.astype(jnp.bfloat16), (((1,), (1,)), ((), ())),
            preferred_element_type=jnp.float32)

    @pl.when(step == 0)
    def _():
        out_ref[...] = mlp(sup_ref[...], sdown_ref[...])

    @pl.when(jnp.logical_and(step > 0, step < N_SHARED_CHUNKS))
    def _():
        out_ref[...] += mlp(sup_ref[...], sdown_ref[...])

    @pl.when(step >= N_SHARED_CHUNKS)
    def _():
        y = mlp(up_ref[0], down_ref[0])
        out_ref[...] += wt_ref[0] * y


def kernel(hidden_states, gate_w, e_score_correction_bias, w_up, w_down,
           shared_w_up, shared_w_down):
    t, d = hidden_states.shape
    e, d_ff, _ = w_up.shape
    d_ff_sh = shared_w_up.shape[0]
    assert e == E and d_ff_sh == N_SHARED_CHUNKS * d_ff
    n_steps = N_SHARED_CHUNKS + E

    x_bf = hidden_states.astype(jnp.bfloat16)
    gw_bf = gate_w.astype(jnp.bfloat16)

    logits_t = pl.pallas_call(
        _logits_kernel,
        out_shape=jax.ShapeDtypeStruct((E, t), jnp.float32),
    )(x_bf, gw_bf)

    tb = t // 128
    lg3 = logits_t.reshape(E, tb, 128)
    bias3 = e_score_correction_bias.reshape(E, 1, 1)
    w3 = pl.pallas_call(
        _routing_kernel,
        out_shape=jax.ShapeDtypeStruct((n_steps, tb, 128), jnp.float32),
    )(lg3, bias3)
    wt3 = w3.reshape(n_steps, t, 1)

    nsc = N_SHARED_CHUNKS
    out = pl.pallas_call(
        _moe_kernel,
        grid=(n_steps,),
        in_specs=[
            pl.BlockSpec((t, d), lambda i: (0, 0)),
            pl.BlockSpec((1, t, 1), lambda i: (i, 0, 0)),
            pl.BlockSpec((1, d_ff, d),
                         lambda i: (jnp.maximum(i - nsc, 0), 0, 0)),
            pl.BlockSpec((1, d, d_ff),
                         lambda i: (jnp.maximum(i - nsc, 0), 0, 0)),
            pl.BlockSpec((d_ff, d), lambda i: (jnp.minimum(i, nsc - 1), 0)),
            pl.BlockSpec((d, d_ff), lambda i: (0, jnp.minimum(i, nsc - 1))),
        ],
        out_specs=pl.BlockSpec((t, d), lambda i: (0, 0)),
        out_shape=jax.ShapeDtypeStruct((t, d), jnp.float32),
        compiler_params=pltpu.CompilerParams(
            dimension_semantics=("arbitrary",),
            vmem_limit_bytes=100 * 1024 * 1024,
        ),
    )(x_bf, wt3, w_up, w_down, shared_w_up, shared_w_down)
    return out


# speedup vs baseline: 1.4923x; 1.0271x over previous
import jax
import jax.numpy as jnp
from jax.experimental import pallas as pl
from jax.experimental.pallas import tpu as pltpu

E = 8
N_GROUP = 4
TOPK_GROUP = 2
TOP_K = 2
GROUP_SIZE = E // N_GROUP
ROUTED_SCALING = 2.5
NEG = -1e30
N_SHARED_CHUNKS = 4


def _logits_kernel(x_ref, gw_ref, out_ref):
    out_ref[...] = jax.lax.dot_general(
        gw_ref[...], x_ref[...], (((1,), (1,)), ((), ())),
        preferred_element_type=jnp.float32)


def _routing_kernel(lg_ref, bias_ref, w_ref):
    sc = [1.0 / (1.0 + jnp.exp(-lg_ref[e])) for e in range(E)]
    sb = [sc[e] + bias_ref[e] for e in range(E)]
    g = [sb[GROUP_SIZE * gi] + sb[GROUP_SIZE * gi + 1] for gi in range(N_GROUP)]
    gmask = []
    for gi in range(N_GROUP):
        r = jnp.zeros_like(g[gi])
        for gj in range(N_GROUP):
            if gj == gi:
                continue
            beats = (g[gj] > g[gi]) if gj > gi else (g[gj] >= g[gi])
            r = r + beats.astype(jnp.float32)
        gmask.append(r < TOPK_GROUP)
    ms = [jnp.where(gmask[e // GROUP_SIZE], sb[e], NEG) for e in range(E)]
    wts = []
    for ei in range(E):
        r = jnp.zeros_like(ms[ei])
        for ej in range(E):
            if ej == ei:
                continue
            beats = (ms[ej] > ms[ei]) if ej > ei else (ms[ej] >= ms[ei])
            r = r + beats.astype(jnp.float32)
        sel = r < TOP_K
        wts.append(jnp.where(sel, sc[ei], 0.0))
    denom = wts[0]
    for e in range(1, E):
        denom = denom + wts[e]
    denom = denom + 1e-20
    ones = jnp.ones_like(wts[0])
    for c in range(N_SHARED_CHUNKS):
        w_ref[c] = ones
    for e in range(E):
        w_ref[N_SHARED_CHUNKS + e] = ROUTED_SCALING * (wts[e] / denom)


def _moe_kernel(x_ref, wt_ref, up_ref, down_ref, sup_ref, sdown_ref, out_ref):
    step = pl.program_id(0)
    t = x_ref.shape[0]
    n_sub = 2
    ts = t // n_sub

    def mlp_acc(wu, wd, first, weighted):
        wub = wu.astype(jnp.bfloat16)
        wdb = wd.astype(jnp.bfloat16)
        for s in range(n_sub):
            sl = pl.ds(s * ts, ts)
            h = jax.lax.dot_general(
                x_ref[sl], wub, (((1,), (1,)), ((), ())),
                preferred_element_type=jnp.float32)
            h = jnp.square(jnp.maximum(h, 0.0)).astype(jnp.bfloat16)
            y = jax.lax.dot_general(
                h, wdb, (((1,), (1,)), ((), ())),
                preferred_element_type=jnp.float32)
            if weighted:
                y = wt_ref[0, sl] * y
            if first:
                out_ref[sl, :] = y
            else:
                out_ref[sl, :] += y

    @pl.when(step == 0)
    def _():
        mlp_acc(sup_ref[...], sdown_ref[...], True, False)

    @pl.when(jnp.logical_and(step > 0, step < N_SHARED_CHUNKS))
    def _():
        mlp_acc(sup_ref[...], sdown_ref[...], False, False)

    @pl.when(step >= N_SHARED_CHUNKS)
    def _():
        mlp_acc(up_ref[0], down_ref[0], False, True)


def kernel(hidden_states, gate_w, e_score_correction_bias, w_up, w_down,
           shared_w_up, shared_w_down):
    t, d = hidden_states.shape
    e, d_ff, _ = w_up.shape
    d_ff_sh = shared_w_up.shape[0]
    assert e == E and d_ff_sh == N_SHARED_CHUNKS * d_ff
    n_steps = N_SHARED_CHUNKS + E

    x_bf = hidden_states.astype(jnp.bfloat16)
    gw_bf = gate_w.astype(jnp.bfloat16)

    logits_t = pl.pallas_call(
        _logits_kernel,
        out_shape=jax.ShapeDtypeStruct((E, t), jnp.float32),
    )(x_bf, gw_bf)

    tb = t // 128
    lg3 = logits_t.reshape(E, tb, 128)
    bias3 = e_score_correction_bias.reshape(E, 1, 1)
    w3 = pl.pallas_call(
        _routing_kernel,
        out_shape=jax.ShapeDtypeStruct((n_steps, tb, 128), jnp.float32),
    )(lg3, bias3)
    wt3 = w3.reshape(n_steps, t, 1)

    nsc = N_SHARED_CHUNKS
    out = pl.pallas_call(
        _moe_kernel,
        grid=(n_steps,),
        in_specs=[
            pl.BlockSpec((t, d), lambda i: (0, 0)),
            pl.BlockSpec((1, t, 1), lambda i: (i, 0, 0)),
            pl.BlockSpec((1, d_ff, d),
                         lambda i: (jnp.maximum(i - nsc, 0), 0, 0)),
            pl.BlockSpec((1, d, d_ff),
                         lambda i: (jnp.maximum(i - nsc, 0), 0, 0)),
            pl.BlockSpec((d_ff, d), lambda i: (jnp.minimum(i, nsc - 1), 0)),
            pl.BlockSpec((d, d_ff), lambda i: (0, jnp.minimum(i, nsc - 1))),
        ],
        out_specs=pl.BlockSpec((t, d), lambda i: (0, 0)),
        out_shape=jax.ShapeDtypeStruct((t, d), jnp.float32),
        compiler_params=pltpu.CompilerParams(
            dimension_semantics=("arbitrary",),
            vmem_limit_bytes=100 * 1024 * 1024,
        ),
    )(x_bf, wt3, w_up, w_down, shared_w_up, shared_w_down)
    return out
